# baseline (speedup 1.0000x reference)
import jax
import jax.numpy as jnp
from jax.experimental import pallas as pl
from jax.experimental.pallas import tpu as pltpu

_B, _L, _D = 2, 4096, 2048
_H, _DH = 16, 128
_TOPK, _WIN = 64, 128
_HALF = _WIN // 2
_LW = _WIN + 2 * _HALF
_NW = _L // _WIN
_SCALE = _DH ** -0.5


def _mm_kernel(x_ref, w_ref, b_ref, o_ref):
    o_ref[...] = (
        jnp.dot(x_ref[...], w_ref[...], preferred_element_type=jnp.float32)
        + b_ref[...]
    )


def _matmul_bias(x2d, w2d, b2d, bm, bn):
    m, k = x2d.shape
    n = w2d.shape[1]
    return pl.pallas_call(
        _mm_kernel,
        grid=(m // bm, n // bn),
        in_specs=[
            pl.BlockSpec((bm, k), lambda i, j: (i, 0)),
            pl.BlockSpec((k, bn), lambda i, j: (0, j)),
            pl.BlockSpec((1, bn), lambda i, j: (0, j)),
        ],
        out_specs=pl.BlockSpec((bm, bn), lambda i, j: (i, j)),
        out_shape=jax.ShapeDtypeStruct((m, n), jnp.float32),
        compiler_params=pltpu.CompilerParams(
            dimension_semantics=("parallel", "parallel")
        ),
    )(x2d, w2d, b2d)


def _topk_kernel(im_ref, idx_ref):
    rows = _L // 128
    v = im_ref[0]
    pos = (
        jax.lax.broadcasted_iota(jnp.int32, (rows, 128), 0) * 128
        + jax.lax.broadcasted_iota(jnp.int32, (rows, 128), 1)
    )
    slot = jax.lax.broadcasted_iota(jnp.int32, (1, _TOPK), 1)

    def body(j, carry):
        vals, res = carry
        m = jnp.max(vals)
        sel = jnp.min(jnp.where(vals == m, pos, _L))
        res = jnp.where(slot == j, sel, res)
        vals = jnp.where(pos == sel, -jnp.inf, vals)
        return vals, res

    _, res = jax.lax.fori_loop(
        0, _TOPK, body, (v, jnp.zeros((1, _TOPK), jnp.int32))
    )
    idx_ref[0] = res


def _topk(im3):
    return pl.pallas_call(
        _topk_kernel,
        grid=(_B,),
        in_specs=[pl.BlockSpec((1, _L // 128, 128), lambda b: (b, 0, 0))],
        out_specs=pl.BlockSpec((1, 1, _TOPK), lambda b: (b, 0, 0)),
        out_shape=jax.ShapeDtypeStruct((_B, 1, _TOPK), jnp.int32),
    )(im3)


def _gather_kernel(idx_ref, src_ref, o_ref):
    del idx_ref
    o_ref[...] = src_ref[...]


def _gather_rows(qkv5, idx_flat):
    out = pl.pallas_call(
        _gather_kernel,
        grid_spec=pltpu.PrefetchScalarGridSpec(
            num_scalar_prefetch=1,
            grid=(2, _B, _TOPK),
            in_specs=[
                pl.BlockSpec(
                    (1, 1, 1, 1, _D),
                    lambda s, b, j, idx: (b, idx[b * _TOPK + j], s + 1, 0, 0),
                )
            ],
            out_specs=pl.BlockSpec(
                (1, 1, 1, 1, _D), lambda s, b, j, idx: (s, b, j, 0, 0)
            ),
        ),
        out_shape=jax.ShapeDtypeStruct((2, _B, _TOPK, 1, _D), jnp.float32),
    )(idx_flat, qkv5)
    return out


def _attn_kernel(pos_ref, q_ref, k_ref, v_ref, kg_ref, vg_ref, o_ref):
    c = pl.program_id(2)
    start = c * _WIN
    ofs = jnp.clip(start - _HALF, 0, _L - _LW)
    q = q_ref[0]
    kl = k_ref[0, pl.ds(ofs, _LW), :]
    vl = v_ref[0, pl.ds(ofs, _LW), :]
    kg = kg_ref[0, 0]
    vg = vg_ref[0, 0]

    sl = jax.lax.dot_general(
        q, kl, (((1,), (1,)), ((), ())), preferred_element_type=jnp.float32
    ) * _SCALE
    sg = jax.lax.dot_general(
        q, kg, (((1,), (1,)), ((), ())), preferred_element_type=jnp.float32
    ) * _SCALE

    q_pos = start + jax.lax.broadcasted_iota(jnp.int32, (_WIN, 1), 0)
    kl_pos = ofs + jax.lax.broadcasted_iota(jnp.int32, (1, _LW), 1)
    local_start = jnp.maximum(start - _HALF, 0)
    vis_l = (kl_pos >= local_start) & (kl_pos <= q_pos)
    vis_g = pos_ref[0] <= q_pos

    sl = jnp.where(vis_l, sl, -jnp.inf)
    sg = jnp.where(vis_g, sg, -jnp.inf)
    m = jnp.maximum(
        jnp.max(sl, axis=1, keepdims=True), jnp.max(sg, axis=1, keepdims=True)
    )
    el = jnp.exp(sl - m)
    eg = jnp.exp(sg - m)
    den = (
        jnp.sum(el, axis=1, keepdims=True)
        + jnp.sum(eg, axis=1, keepdims=True)
    )
    ol = jax.lax.dot_general(
        el, vl, (((1,), (0,)), ((), ())), preferred_element_type=jnp.float32
    )
    og = jax.lax.dot_general(
        eg, vg, (((1,), (0,)), ((), ())), preferred_element_type=jnp.float32
    )
    o_ref[0] = (ol + og) / den


def _attention(qkv3, kvg, pos3):
    return pl.pallas_call(
        _attn_kernel,
        grid=(_B, _H, _NW),
        in_specs=[
            pl.BlockSpec((1, 1, _TOPK), lambda b, h, c: (0, 0, 0)),
            pl.BlockSpec((1, _WIN, _DH), lambda b, h, c: (b, c, h)),
            pl.BlockSpec((1, _L, _DH), lambda b, h, c: (b, 0, _H + h)),
            pl.BlockSpec((1, _L, _DH), lambda b, h, c: (b, 0, 2 * _H + h)),
            pl.BlockSpec((1, 1, _TOPK, _DH), lambda b, h, c: (0, b, 0, h)),
            pl.BlockSpec((1, 1, _TOPK, _DH), lambda b, h, c: (1, b, 0, h)),
        ],
        out_specs=pl.BlockSpec((1, _WIN, _DH), lambda b, h, c: (b, c, h)),
        out_shape=jax.ShapeDtypeStruct((_B, _L, _D), jnp.float32),
        compiler_params=pltpu.CompilerParams(
            dimension_semantics=("parallel", "parallel", "arbitrary")
        ),
    )(pos3, qkv3, qkv3, qkv3, kvg, kvg)


def kernel(x, importance_mask, Wq, bq, Wk, bk, Wv, bv, Wo, bo):
    xf = x.reshape(_B * _L, _D)
    wqkv = jnp.concatenate([Wq.T, Wk.T, Wv.T], axis=1)
    bqkv = jnp.concatenate([bq, bk, bv]).reshape(1, 3 * _D)
    qkv = _matmul_bias(xf, wqkv, bqkv, bm=512, bn=768)
    qkv3 = qkv.reshape(_B, _L, 3 * _D)

    im3 = importance_mask.reshape(_B, _L // 128, 128)
    topk = _topk(im3)
    idx_flat = topk.reshape(_B * _TOPK)
    kvg = _gather_rows(qkv3.reshape(_B, _L, 3, 1, _D), idx_flat)
    kvg = kvg.reshape(2, _B, _TOPK, _D)

    attn = _attention(qkv3, kvg, topk[0:1])
    out = _matmul_bias(
        attn.reshape(_B * _L, _D), Wo.T, bo.reshape(1, _D),
        bm=512, bn=min(512, _D),
    )
    return out.reshape(_B, _L, _D)


# speedup vs baseline: 2.8030x; 2.8030x over previous
import functools

import jax
import jax.numpy as jnp
from jax import lax
from jax.experimental import pallas as pl
from jax.experimental.pallas import tpu as pltpu
from jax.experimental.pallas import tpu_sc as plsc

_B, _L, _D = 2, 4096, 2048
_H, _DH = 16, 128
_TOPK, _WIN = 64, 128
_HALF = _WIN // 2
_NW = _L // _WIN
_SCALE = _DH ** -0.5

_MQ = 256
_NKV = _MQ // 2 + _MQ + _TOPK


def _mm_t_kernel(x_ref, w_ref, b_ref, o_ref):
    o_ref[...] = lax.dot_general(
        x_ref[...], w_ref[...], (((1,), (1,)), ((), ())),
        preferred_element_type=jnp.float32,
    ) + b_ref[...]


def _matmul_t_bias(x2d, w, b1d, bm):
    m, k = x2d.shape
    n = w.shape[0]
    return pl.pallas_call(
        _mm_t_kernel,
        grid=(m // bm,),
        in_specs=[
            pl.BlockSpec((bm, k), lambda i: (i, 0)),
            pl.BlockSpec((n, k), lambda i: (0, 0)),
            pl.BlockSpec((1, n), lambda i: (0, 0)),
        ],
        out_specs=pl.BlockSpec((bm, n), lambda i: (i, 0)),
        out_shape=jax.ShapeDtypeStruct((m, n), jnp.float32),
        compiler_params=pltpu.CompilerParams(
            dimension_semantics=("arbitrary",)
        ),
    )(x2d, w, b1d.reshape(1, n))


def _topk_kernel(im_ref, idx_ref):
    rows = _L // 128
    v = im_ref[0]
    pos = (
        jax.lax.broadcasted_iota(jnp.int32, (rows, 128), 0) * 128
        + jax.lax.broadcasted_iota(jnp.int32, (rows, 128), 1)
    )
    slot = jax.lax.broadcasted_iota(jnp.int32, (1, _TOPK), 1)

    def body(j, carry):
        vals, res = carry
        m = jnp.max(vals)
        sel = jnp.min(jnp.where(vals == m, pos, _L))
        res = jnp.where(slot == j, sel, res)
        vals = jnp.where(pos == sel, -jnp.inf, vals)
        return vals, res

    _, res = jax.lax.fori_loop(
        0, _TOPK, body, (v, jnp.zeros((1, _TOPK), jnp.int32))
    )
    idx_ref[0] = res


def _topk(im3):
    return pl.pallas_call(
        _topk_kernel,
        grid=(_B,),
        in_specs=[pl.BlockSpec((1, _L // 128, 128), lambda b: (b, 0, 0))],
        out_specs=pl.BlockSpec((1, 1, _TOPK), lambda b: (b, 0, 0)),
        out_shape=jax.ShapeDtypeStruct((_B, 1, _TOPK), jnp.int32),
    )(im3)


def _sc_gather_rows(x2d, idx_abs):
    nrows = _B * _TOPK
    per_w = 8
    nw_used = nrows // per_w
    mesh = plsc.VectorSubcoreMesh(core_axis_name="c", subcore_axis_name="s")

    @functools.partial(
        pl.kernel, mesh=mesh,
        out_type=jax.ShapeDtypeStruct((nrows, _D), jnp.float32),
        scratch_types=[
            pltpu.VMEM((per_w,), jnp.int32),
            pltpu.VMEM((per_w, _D), jnp.float32),
            pltpu.SemaphoreType.DMA,
        ],
    )
    def k(x_hbm, idx_hbm, out_hbm, idx_v, rows_v, sem):
        wid = lax.axis_index("s") * 2 + lax.axis_index("c")

        @pl.when(wid < nw_used)
        def _():
            base = wid * per_w
            pltpu.sync_copy(idx_hbm.at[pl.ds(base, per_w)], idx_v)
            pltpu.async_copy(x_hbm.at[idx_v], rows_v, sem).wait()
            pltpu.sync_copy(rows_v, out_hbm.at[pl.ds(base, per_w)])

    return k(x2d, idx_abs)


def _attn_kernel(pos_ref, q_ref, kp_ref, kc_ref, vp_ref, vc_ref,
                 kg_ref, vg_ref, wo_ref, bo_ref, o_ref):
    m_i = pl.program_id(1)
    q0 = m_i * _MQ
    ph0 = jnp.maximum(m_i - 1, 0) * _MQ + _MQ // 2

    colh = jax.lax.broadcasted_iota(jnp.int32, (1, _MQ // 2), 1)
    col = jax.lax.broadcasted_iota(jnp.int32, (1, _MQ), 1)
    q_pos = q0 + jax.lax.broadcasted_iota(jnp.int32, (_MQ, 1), 0)
    kv_pos = jnp.concatenate(
        [ph0 + colh, q0 + col, pos_ref[0]], axis=1)
    nkcol = jax.lax.broadcasted_iota(jnp.int32, (1, _NKV), 1)
    is_prev = nkcol < _MQ // 2
    is_glob = nkcol >= _MQ // 2 + _MQ
    win_start = (q_pos // _WIN) * _WIN
    local_ok = (kv_pos >= win_start - _HALF) & ((~is_prev) | (kv_pos < q0))
    vis = (kv_pos <= q_pos) & (is_glob | local_ok)
    mask_add = jnp.where(vis, 0.0, -jnp.inf)

    acc = jnp.zeros((_MQ, _D), jnp.float32) + bo_ref[...]
    for h in range(_H):
        hs = slice(h * _DH, (h + 1) * _DH)
        q = q_ref[0, :, hs] * _SCALE
        kcat = jnp.concatenate(
            [kp_ref[0, _MQ // 2:, hs], kc_ref[0, :, hs],
             kg_ref[0, :, 0, hs]], axis=0)
        vcat = jnp.concatenate(
            [vp_ref[0, _MQ // 2:, hs], vc_ref[0, :, hs],
             vg_ref[0, :, 0, hs]], axis=0)
        s = lax.dot_general(
            q, kcat, (((1,), (1,)), ((), ())),
            preferred_element_type=jnp.float32,
        ) + mask_add
        mx = jnp.max(s, axis=1, keepdims=True)
        e = jnp.exp(s - mx)
        den = jnp.sum(e, axis=1, keepdims=True)
        o = lax.dot_general(
            e, vcat, (((1,), (0,)), ((), ())),
            preferred_element_type=jnp.float32,
        ) / den
        acc = acc + lax.dot_general(
            o, wo_ref[:, hs], (((1,), (1,)), ((), ())),
            preferred_element_type=jnp.float32,
        )
    o_ref[0] = acc


def _attention(q3, k3, v3, kg4, vg4, pos3, wo, bo):
    nm = _L // _MQ
    return pl.pallas_call(
        _attn_kernel,
        grid=(_B, nm),
        in_specs=[
            pl.BlockSpec((1, 1, _TOPK), lambda b, m: (0, 0, 0)),
            pl.BlockSpec((1, _MQ, _D), lambda b, m: (b, m, 0)),
            pl.BlockSpec((1, _MQ, _D), lambda b, m: (b, jnp.maximum(m - 1, 0), 0)),
            pl.BlockSpec((1, _MQ, _D), lambda b, m: (b, m, 0)),
            pl.BlockSpec((1, _MQ, _D), lambda b, m: (b, jnp.maximum(m - 1, 0), 0)),
            pl.BlockSpec((1, _MQ, _D), lambda b, m: (b, m, 0)),
            pl.BlockSpec((1, _TOPK, 1, _D), lambda b, m: (b, 0, 0, 0)),
            pl.BlockSpec((1, _TOPK, 1, _D), lambda b, m: (b, 0, 0, 0)),
            pl.BlockSpec((_D, _D), lambda b, m: (0, 0)),
            pl.BlockSpec((1, _D), lambda b, m: (0, 0)),
        ],
        out_specs=pl.BlockSpec((1, _MQ, _D), lambda b, m: (b, m, 0)),
        out_shape=jax.ShapeDtypeStruct((_B, _L, _D), jnp.float32),
        compiler_params=pltpu.CompilerParams(
            dimension_semantics=("parallel", "arbitrary")
        ),
    )(pos3, q3, k3, k3, v3, v3, kg4, vg4, wo, bo.reshape(1, _D))


def kernel(x, importance_mask, Wq, bq, Wk, bk, Wv, bv, Wo, bo):
    xf = x.reshape(_B * _L, _D)
    q2 = _matmul_t_bias(xf, Wq, bq, bm=512)
    k2 = _matmul_t_bias(xf, Wk, bk, bm=512)
    v2 = _matmul_t_bias(xf, Wv, bv, bm=512)

    im3 = importance_mask.reshape(_B, _L // 128, 128)
    topk = _topk(im3)
    idx_abs = (
        topk.reshape(_B, _TOPK)
        + (jnp.arange(_B, dtype=jnp.int32) * _L)[:, None]
    ).reshape(_B * _TOPK)
    xg = _sc_gather_rows(xf, idx_abs)
    kg2 = _matmul_t_bias(xg, Wk, bk, bm=_B * _TOPK)
    vg2 = _matmul_t_bias(xg, Wv, bv, bm=_B * _TOPK)

    out = _attention(
        q2.reshape(_B, _L, _D), k2.reshape(_B, _L, _D),
        v2.reshape(_B, _L, _D),
        kg2.reshape(_B, _TOPK, 1, _D), vg2.reshape(_B, _TOPK, 1, _D),
        topk[0:1], Wo, bo,
    )
    return out


# speedup vs baseline: 3.2886x; 1.1732x over previous
import functools

import jax
import jax.numpy as jnp
from jax import lax
from jax.experimental import pallas as pl
from jax.experimental.pallas import tpu as pltpu
from jax.experimental.pallas import tpu_sc as plsc

_B, _L, _D = 2, 4096, 2048
_H, _DH = 16, 128
_TOPK, _WIN = 64, 128
_HALF = _WIN // 2
_NW = _L // _WIN
_SCALE = _DH ** -0.5

_MQ = 256
_NKV = _MQ // 2 + _MQ + _TOPK


def _qkv_kernel(x_ref, wq_ref, wk_ref, wv_ref, bq_ref, bk_ref, bv_ref,
                q_ref, k_ref, v_ref):
    xv = x_ref[...]

    def proj(w_ref, b_ref):
        return lax.dot_general(
            xv, w_ref[...], (((1,), (1,)), ((), ())),
            preferred_element_type=jnp.float32,
        ) + b_ref[...]

    q_ref[...] = (proj(wq_ref, bq_ref) * _SCALE).astype(jnp.bfloat16)
    k_ref[...] = proj(wk_ref, bk_ref).astype(jnp.bfloat16)
    v_ref[...] = proj(wv_ref, bv_ref).astype(jnp.bfloat16)


def _qkv_proj(xbf, wq, wk, wv, bq, bk, bv, bm):
    m, k = xbf.shape
    n = wq.shape[0]
    wspec = pl.BlockSpec((n, k), lambda i: (0, 0))
    bspec = pl.BlockSpec((1, n), lambda i: (0, 0))
    ospec = pl.BlockSpec((bm, n), lambda i: (i, 0))
    oshape = jax.ShapeDtypeStruct((m, n), jnp.bfloat16)
    return pl.pallas_call(
        _qkv_kernel,
        grid=(m // bm,),
        in_specs=[pl.BlockSpec((bm, k), lambda i: (i, 0)),
                  wspec, wspec, wspec, bspec, bspec, bspec],
        out_specs=[ospec, ospec, ospec],
        out_shape=[oshape, oshape, oshape],
        compiler_params=pltpu.CompilerParams(
            dimension_semantics=("parallel",)
        ),
    )(xbf, wq, wk, wv, bq.reshape(1, n), bk.reshape(1, n), bv.reshape(1, n))


def _kv_glob_kernel(x_ref, wk_ref, wv_ref, bk_ref, bv_ref, k_ref, v_ref):
    xv = x_ref[...]

    def proj(w_ref, b_ref):
        return lax.dot_general(
            xv, w_ref[...], (((1,), (1,)), ((), ())),
            preferred_element_type=jnp.float32,
        ) + b_ref[...]

    k_ref[...] = proj(wk_ref, bk_ref).astype(jnp.bfloat16)
    v_ref[...] = proj(wv_ref, bv_ref).astype(jnp.bfloat16)


def _kv_glob_proj(xgbf, wk, wv, bk, bv):
    m, k = xgbf.shape
    n = wk.shape[0]
    wspec = pl.BlockSpec((n, k), lambda: (0, 0))
    bspec = pl.BlockSpec((1, n), lambda: (0, 0))
    ospec = pl.BlockSpec((m, n), lambda: (0, 0))
    oshape = jax.ShapeDtypeStruct((m, n), jnp.bfloat16)
    return pl.pallas_call(
        _kv_glob_kernel,
        in_specs=[pl.BlockSpec((m, k), lambda: (0, 0)),
                  wspec, wspec, bspec, bspec],
        out_specs=[ospec, ospec],
        out_shape=[oshape, oshape],
    )(xgbf, wk, wv, bk.reshape(1, n), bv.reshape(1, n))


def _topk_kernel(im_ref, idx_ref):
    rows = _L // 128
    v = im_ref[0]
    pos = (
        jax.lax.broadcasted_iota(jnp.int32, (rows, 128), 0) * 128
        + jax.lax.broadcasted_iota(jnp.int32, (rows, 128), 1)
    )
    slot = jax.lax.broadcasted_iota(jnp.int32, (1, _TOPK), 1)

    boff = pl.program_id(0) * _L

    def body(j, carry):
        vals, res = carry
        m = jnp.max(vals)
        sel = jnp.min(jnp.where(vals == m, pos, _L))
        res = jnp.where(slot == j, sel + boff, res)
        vals = jnp.where(pos == sel, -jnp.inf, vals)
        return vals, res

    _, res = jax.lax.fori_loop(
        0, _TOPK, body, (v, jnp.zeros((1, _TOPK), jnp.int32))
    )
    idx_ref[0] = res


def _topk(im3):
    return pl.pallas_call(
        _topk_kernel,
        grid=(_B,),
        in_specs=[pl.BlockSpec((1, _L // 128, 128), lambda b: (b, 0, 0))],
        out_specs=pl.BlockSpec((1, 1, _TOPK), lambda b: (b, 0, 0)),
        out_shape=jax.ShapeDtypeStruct((_B, 1, _TOPK), jnp.int32),
    )(im3)


def _sc_gather_rows(x2d, idx_abs):
    nrows = _B * _TOPK
    per_w = 8
    nw_used = nrows // per_w
    mesh = plsc.VectorSubcoreMesh(core_axis_name="c", subcore_axis_name="s")

    @functools.partial(
        pl.kernel, mesh=mesh,
        out_type=jax.ShapeDtypeStruct((nrows, _D), jnp.float32),
        scratch_types=[
            pltpu.VMEM((per_w,), jnp.int32),
            pltpu.VMEM((per_w, _D), jnp.float32),
            pltpu.SemaphoreType.DMA,
        ],
    )
    def k(x_hbm, idx_hbm, out_hbm, idx_v, rows_v, sem):
        wid = lax.axis_index("s") * 2 + lax.axis_index("c")

        @pl.when(wid < nw_used)
        def _():
            base = wid * per_w
            pltpu.sync_copy(idx_hbm.at[pl.ds(base, per_w)], idx_v)
            pltpu.async_copy(x_hbm.at[idx_v], rows_v, sem).wait()
            pltpu.sync_copy(rows_v, out_hbm.at[pl.ds(base, per_w)])

    return k(x2d, idx_abs)


def _attn_kernel(pos_ref, q_ref, kp_ref, kc_ref, vp_ref, vc_ref,
                 kg_ref, vg_ref, wo_ref, bo_ref, o_ref):
    m_i = pl.program_id(1)
    q0 = m_i * _MQ
    ph0 = jnp.maximum(m_i - 1, 0) * _MQ + _MQ // 2

    colh = jax.lax.broadcasted_iota(jnp.int32, (1, _MQ // 2), 1)
    col = jax.lax.broadcasted_iota(jnp.int32, (1, _MQ), 1)
    q_pos = q0 + jax.lax.broadcasted_iota(jnp.int32, (_MQ, 1), 0)
    kv_pos = jnp.concatenate(
        [ph0 + colh, q0 + col, pos_ref[0]], axis=1)
    nkcol = jax.lax.broadcasted_iota(jnp.int32, (1, _NKV), 1)
    is_prev = nkcol < _MQ // 2
    is_glob = nkcol >= _MQ // 2 + _MQ
    win_start = (q_pos // _WIN) * _WIN
    local_ok = (kv_pos >= win_start - _HALF) & ((~is_prev) | (kv_pos < q0))
    vis = (kv_pos <= q_pos) & (is_glob | local_ok)
    mask_add = jnp.where(vis, 0.0, -jnp.inf)

    outs = []
    for h in range(_H):
        hs = slice(h * _DH, (h + 1) * _DH)
        q = q_ref[0, :, hs]
        kcat = jnp.concatenate(
            [kp_ref[0, _MQ // 2:, hs], kc_ref[0, :, hs],
             kg_ref[0, :, 0, hs]], axis=0)
        vcat = jnp.concatenate(
            [vp_ref[0, _MQ // 2:, hs], vc_ref[0, :, hs],
             vg_ref[0, :, 0, hs]], axis=0)
        s = lax.dot_general(
            q, kcat, (((1,), (1,)), ((), ())),
            preferred_element_type=jnp.float32,
        ) + mask_add
        mx = jnp.max(s, axis=1, keepdims=True)
        e = jnp.exp(s - mx)
        den = jnp.sum(e, axis=1, keepdims=True)
        o = lax.dot_general(
            e.astype(jnp.bfloat16), vcat, (((1,), (0,)), ((), ())),
            preferred_element_type=jnp.float32,
        ) / den
        outs.append(o.astype(jnp.bfloat16))
    o_all = jnp.concatenate(outs, axis=1)
    o_ref[0] = lax.dot_general(
        o_all, wo_ref[...], (((1,), (1,)), ((), ())),
        preferred_element_type=jnp.float32,
    ) + bo_ref[...]


def _attention(q3, k3, v3, kg4, vg4, pos3, wo, bo):
    nm = _L // _MQ
    return pl.pallas_call(
        _attn_kernel,
        grid=(_B, nm),
        in_specs=[
            pl.BlockSpec((1, 1, _TOPK), lambda b, m: (0, 0, 0)),
            pl.BlockSpec((1, _MQ, _D), lambda b, m: (b, m, 0)),
            pl.BlockSpec((1, _MQ, _D), lambda b, m: (b, jnp.maximum(m - 1, 0), 0)),
            pl.BlockSpec((1, _MQ, _D), lambda b, m: (b, m, 0)),
            pl.BlockSpec((1, _MQ, _D), lambda b, m: (b, jnp.maximum(m - 1, 0), 0)),
            pl.BlockSpec((1, _MQ, _D), lambda b, m: (b, m, 0)),
            pl.BlockSpec((1, _TOPK, 1, _D), lambda b, m: (b, 0, 0, 0)),
            pl.BlockSpec((1, _TOPK, 1, _D), lambda b, m: (b, 0, 0, 0)),
            pl.BlockSpec((_D, _D), lambda b, m: (0, 0)),
            pl.BlockSpec((1, _D), lambda b, m: (0, 0)),
        ],
        out_specs=pl.BlockSpec((1, _MQ, _D), lambda b, m: (b, m, 0)),
        out_shape=jax.ShapeDtypeStruct((_B, _L, _D), jnp.float32),
        compiler_params=pltpu.CompilerParams(
            dimension_semantics=("parallel", "arbitrary")
        ),
    )(pos3, q3, k3, k3, v3, v3, kg4, vg4, wo, bo.reshape(1, _D))


def kernel(x, importance_mask, Wq, bq, Wk, bk, Wv, bv, Wo, bo):
    xf = x.reshape(_B * _L, _D)
    xbf = xf.astype(jnp.bfloat16)
    wq, wk, wv, wo = (w.astype(jnp.bfloat16) for w in (Wq, Wk, Wv, Wo))
    q2, k2, v2 = _qkv_proj(xbf, wq, wk, wv, bq, bk, bv, bm=512)

    im3 = importance_mask.reshape(_B, _L // 128, 128)
    topk = _topk(im3)
    idx_abs = topk.reshape(_B * _TOPK)
    xg = _sc_gather_rows(xf, idx_abs)
    kg2, vg2 = _kv_glob_proj(xg.astype(jnp.bfloat16), wk, wv, bk, bv)

    out = _attention(
        q2.reshape(_B, _L, _D), k2.reshape(_B, _L, _D),
        v2.reshape(_B, _L, _D),
        kg2.reshape(_B, _TOPK, 1, _D), vg2.reshape(_B, _TOPK, 1, _D),
        topk[0:1], wo, bo,
    )
    return out


# speedup vs baseline: 3.3729x; 1.0256x over previous
import dataclasses
import functools

import jax
import jax.numpy as jnp
from jax import lax
from jax.experimental import pallas as pl
from jax.experimental.pallas import tpu as pltpu
from jax.experimental.pallas import tpu_sc as plsc

_B, _L, _D = 2, 4096, 2048
_H, _DH = 16, 128
_TOPK, _WIN = 64, 128
_HALF = _WIN // 2
_NW = _L // _WIN
_SCALE = _DH ** -0.5

_MQ = 256
_NKV = _MQ // 2 + _MQ + _TOPK


def _qkv_kernel(x_ref, wq_ref, wk_ref, wv_ref, bq_ref, bk_ref, bv_ref,
                q_ref, k_ref, v_ref):
    xv = x_ref[...]

    def proj(w_ref, b_ref):
        return lax.dot_general(
            xv, w_ref[...], (((1,), (1,)), ((), ())),
            preferred_element_type=jnp.float32,
        ) + b_ref[...]

    q_ref[...] = (proj(wq_ref, bq_ref) * _SCALE).astype(jnp.bfloat16)
    k_ref[...] = proj(wk_ref, bk_ref).astype(jnp.bfloat16)
    v_ref[...] = proj(wv_ref, bv_ref).astype(jnp.bfloat16)


def _qkv_proj(xbf, wq, wk, wv, bq, bk, bv, bm):
    m, k = xbf.shape
    n = wq.shape[0]
    wspec = pl.BlockSpec((n, k), lambda i: (0, 0))
    bspec = pl.BlockSpec((1, n), lambda i: (0, 0))
    ospec = pl.BlockSpec((bm, n), lambda i: (i, 0))
    oshape = jax.ShapeDtypeStruct((m, n), jnp.bfloat16)
    return pl.pallas_call(
        _qkv_kernel,
        grid=(m // bm,),
        in_specs=[pl.BlockSpec((bm, k), lambda i: (i, 0)),
                  wspec, wspec, wspec, bspec, bspec, bspec],
        out_specs=[ospec, ospec, ospec],
        out_shape=[oshape, oshape, oshape],
        compiler_params=pltpu.CompilerParams(
            dimension_semantics=("parallel",)
        ),
    )(xbf, wq, wk, wv, bq.reshape(1, n), bk.reshape(1, n), bv.reshape(1, n))


def _kv_glob_kernel(x_ref, wk_ref, wv_ref, bk_ref, bv_ref, k_ref, v_ref):
    xv = x_ref[...]

    def proj(w_ref, b_ref):
        return lax.dot_general(
            xv, w_ref[...], (((1,), (1,)), ((), ())),
            preferred_element_type=jnp.float32,
        ) + b_ref[...]

    k_ref[...] = proj(wk_ref, bk_ref).astype(jnp.bfloat16)
    v_ref[...] = proj(wv_ref, bv_ref).astype(jnp.bfloat16)


def _kv_glob_proj(xgbf, wk, wv, bk, bv):
    m, k = xgbf.shape
    n = wk.shape[0]
    wspec = pl.BlockSpec((n, k), lambda: (0, 0))
    bspec = pl.BlockSpec((1, n), lambda: (0, 0))
    ospec = pl.BlockSpec((m, n), lambda: (0, 0))
    oshape = jax.ShapeDtypeStruct((m, n), jnp.bfloat16)
    return pl.pallas_call(
        _kv_glob_kernel,
        in_specs=[pl.BlockSpec((m, k), lambda: (0, 0)),
                  wspec, wspec, bspec, bspec],
        out_specs=[ospec, ospec],
        out_shape=[oshape, oshape],
    )(xgbf, wk, wv, bk.reshape(1, n), bv.reshape(1, n))


def _topk_kernel(im_ref, idx_ref):
    rows = _L // 128
    v = im_ref[0]
    pos = (
        jax.lax.broadcasted_iota(jnp.int32, (rows, 128), 0) * 128
        + jax.lax.broadcasted_iota(jnp.int32, (rows, 128), 1)
    )
    slot = jax.lax.broadcasted_iota(jnp.int32, (1, _TOPK), 1)

    boff = pl.program_id(0) * _L

    def body(j, carry):
        vals, res = carry
        m = jnp.max(vals)
        sel = jnp.min(jnp.where(vals == m, pos, _L))
        res = jnp.where(slot == j, sel + boff, res)
        vals = jnp.where(pos == sel, -jnp.inf, vals)
        return vals, res

    _, res = jax.lax.fori_loop(
        0, _TOPK, body, (v, jnp.zeros((1, _TOPK), jnp.int32))
    )
    idx_ref[0] = res


def _topk(im3):
    return pl.pallas_call(
        _topk_kernel,
        grid=(_B,),
        in_specs=[pl.BlockSpec((1, _L // 128, 128), lambda b: (b, 0, 0))],
        out_specs=pl.BlockSpec((1, 1, _TOPK), lambda b: (b, 0, 0)),
        out_shape=jax.ShapeDtypeStruct((_B, 1, _TOPK), jnp.int32),
    )(im3)


_NEG = float("-inf")


def _sc_topk(imf):
    ch = _L // 16
    nv = ch // 16
    mesh = plsc.VectorSubcoreMesh(core_axis_name="c", subcore_axis_name="s")
    cp = pltpu.CompilerParams()
    if "needs_layout_passes" in pltpu.CompilerParams.__dataclass_fields__:
        cp = dataclasses.replace(cp, needs_layout_passes=False)

    @functools.partial(
        pl.kernel, mesh=mesh, compiler_params=cp,
        out_type=jax.ShapeDtypeStruct((_B * _TOPK,), jnp.int32),
        scratch_types=[
            pltpu.VMEM((ch,), jnp.float32),
            pltpu.VMEM((_TOPK,), jnp.float32),
            pltpu.VMEM((_TOPK,), jnp.int32),
            pltpu.VMEM((16 * _TOPK,), jnp.float32),
            pltpu.VMEM((16 * _TOPK,), jnp.int32),
            pltpu.VMEM((_TOPK,), jnp.int32),
            pltpu.VMEM_SHARED((16 * _TOPK,), jnp.float32),
            pltpu.VMEM_SHARED((16 * _TOPK,), jnp.int32),
        ],
    )
    def k(im_hbm, out_hbm, chunk, lv, li, mv, mi, res, shv, shi):
        b = lax.axis_index("c")
        woff = lax.axis_index("s")
        base = b * _L + woff * ch
        pltpu.sync_copy(im_hbm.at[pl.ds(base, ch)], chunk)
        iota = lax.iota(jnp.int32, 16)
        big = jnp.int32(_B * _L)

        vs0 = tuple(chunk[pl.ds(i * 16, 16)] for i in range(nv))

        def p1_body(j, carry):
            vs = carry[0]
            rv = carry[1]
            ri = carry[2]
            m = vs[0]
            for i in range(1, nv):
                m = jnp.maximum(m, vs[i])
            mx = lax.reduce_max(m, axes=(0,))
            cand = jnp.where(vs[0] == mx, iota + (base + 0 * 16), big)
            for i in range(1, nv):
                cand = jnp.minimum(
                    cand, jnp.where(vs[i] == mx, iota + (base + i * 16), big))
            sel = lax.reduce_min(cand, axes=(0,))
            vs = tuple(
                jnp.where(iota + (base + i * 16) == sel, _NEG, vs[i])
                for i in range(nv))
            rv = tuple(
                jnp.where(iota + r * 16 == j, mx, rv[r])
                for r in range(_TOPK // 16))
            ri = tuple(
                jnp.where(iota + r * 16 == j, sel, ri[r])
                for r in range(_TOPK // 16))
            return (vs, rv, ri)

        z4f = tuple(jnp.zeros((16,), jnp.float32) for _ in range(_TOPK // 16))
        z4i = tuple(jnp.zeros((16,), jnp.int32) for _ in range(_TOPK // 16))
        _, rv, ri = lax.fori_loop(0, _TOPK, p1_body, (vs0, z4f, z4i))
        for r in range(_TOPK // 16):
            lv[pl.ds(r * 16, 16)] = rv[r]
            li[pl.ds(r * 16, 16)] = ri[r]
        pltpu.sync_copy(lv, shv.at[pl.ds(woff * _TOPK, _TOPK)])
        pltpu.sync_copy(li, shi.at[pl.ds(woff * _TOPK, _TOPK)])
        plsc.subcore_barrier()

        @pl.when(woff == 0)
        def _():
            pltpu.sync_copy(shv, mv)
            pltpu.sync_copy(shi, mi)
            nmv = 16 * _TOPK // 16

            def p2_body(j, _):
                m = mv[pl.ds(0, 16)]
                for i in range(1, nmv):
                    m = jnp.maximum(m, mv[pl.ds(i * 16, 16)])
                mx = lax.reduce_max(m, axes=(0,))
                cand = jnp.where(mv[pl.ds(0, 16)] == mx, mi[pl.ds(0, 16)], big)
                for i in range(1, nmv):
                    cand = jnp.minimum(
                        cand,
                        jnp.where(mv[pl.ds(i * 16, 16)] == mx,
                                  mi[pl.ds(i * 16, 16)], big))
                sel = lax.reduce_min(cand, axes=(0,))
                for i in range(nmv):
                    vv = mv[pl.ds(i * 16, 16)]
                    mv[pl.ds(i * 16, 16)] = jnp.where(
                        mi[pl.ds(i * 16, 16)] == sel, _NEG, vv)
                res[pl.ds(0, 16)] = jnp.where(
                    iota == j, sel, res[pl.ds(0, 16)])
                for r in range(1, _TOPK // 16):
                    res[pl.ds(r * 16, 16)] = jnp.where(
                        iota + r * 16 == j, sel, res[pl.ds(r * 16, 16)])
                return 0

            lax.fori_loop(0, _TOPK, p2_body, 0)
            pltpu.sync_copy(res, out_hbm.at[pl.ds(b * _TOPK, _TOPK)])

    return k(imf)


def _sc_gather_rows(x2d, idx_abs):
    nrows = _B * _TOPK
    per_w = 8
    nw_used = nrows // per_w
    mesh = plsc.VectorSubcoreMesh(core_axis_name="c", subcore_axis_name="s")

    @functools.partial(
        pl.kernel, mesh=mesh,
        out_type=jax.ShapeDtypeStruct((nrows, _D), jnp.float32),
        scratch_types=[
            pltpu.VMEM((per_w,), jnp.int32),
            pltpu.VMEM((per_w, _D), jnp.float32),
            pltpu.SemaphoreType.DMA,
        ],
    )
    def k(x_hbm, idx_hbm, out_hbm, idx_v, rows_v, sem):
        wid = lax.axis_index("s") * 2 + lax.axis_index("c")

        @pl.when(wid < nw_used)
        def _():
            base = wid * per_w
            pltpu.sync_copy(idx_hbm.at[pl.ds(base, per_w)], idx_v)
            pltpu.async_copy(x_hbm.at[idx_v], rows_v, sem).wait()
            pltpu.sync_copy(rows_v, out_hbm.at[pl.ds(base, per_w)])

    return k(x2d, idx_abs)


def _attn_kernel(pos_ref, q_ref, kp_ref, kc_ref, vp_ref, vc_ref,
                 kg_ref, vg_ref, wo_ref, bo_ref, o_ref):
    m_i = pl.program_id(1)
    q0 = m_i * _MQ
    ph0 = jnp.maximum(m_i - 1, 0) * _MQ + _MQ // 2

    colh = jax.lax.broadcasted_iota(jnp.int32, (1, _MQ // 2), 1)
    col = jax.lax.broadcasted_iota(jnp.int32, (1, _MQ), 1)
    q_pos = q0 + jax.lax.broadcasted_iota(jnp.int32, (_MQ, 1), 0)
    kv_pos = jnp.concatenate(
        [ph0 + colh, q0 + col, pos_ref[0]], axis=1)
    nkcol = jax.lax.broadcasted_iota(jnp.int32, (1, _NKV), 1)
    is_prev = nkcol < _MQ // 2
    is_glob = nkcol >= _MQ // 2 + _MQ
    win_start = (q_pos // _WIN) * _WIN
    local_ok = (kv_pos >= win_start - _HALF) & ((~is_prev) | (kv_pos < q0))
    vis = (kv_pos <= q_pos) & (is_glob | local_ok)
    mask_add = jnp.where(vis, 0.0, -jnp.inf)

    outs = []
    for h in range(_H):
        hs = slice(h * _DH, (h + 1) * _DH)
        q = q_ref[0, :, hs]
        kcat = jnp.concatenate(
            [kp_ref[0, _MQ // 2:, hs], kc_ref[0, :, hs],
             kg_ref[0, :, 0, hs]], axis=0)
        vcat = jnp.concatenate(
            [vp_ref[0, _MQ // 2:, hs], vc_ref[0, :, hs],
             vg_ref[0, :, 0, hs]], axis=0)
        s = lax.dot_general(
            q, kcat, (((1,), (1,)), ((), ())),
            preferred_element_type=jnp.float32,
        ) + mask_add
        mx = jnp.max(s, axis=1, keepdims=True)
        e = jnp.exp(s - mx)
        den = jnp.sum(e, axis=1, keepdims=True)
        o = lax.dot_general(
            e.astype(jnp.bfloat16), vcat, (((1,), (0,)), ((), ())),
            preferred_element_type=jnp.float32,
        ) / den
        outs.append(o.astype(jnp.bfloat16))
    o_all = jnp.concatenate(outs, axis=1)
    o_ref[0] = lax.dot_general(
        o_all, wo_ref[...], (((1,), (1,)), ((), ())),
        preferred_element_type=jnp.float32,
    ) + bo_ref[...]


def _attention(q3, k3, v3, kg4, vg4, pos3, wo, bo):
    nm = _L // _MQ
    return pl.pallas_call(
        _attn_kernel,
        grid=(_B, nm),
        in_specs=[
            pl.BlockSpec((1, 1, _TOPK), lambda b, m: (0, 0, 0)),
            pl.BlockSpec((1, _MQ, _D), lambda b, m: (b, m, 0)),
            pl.BlockSpec((1, _MQ, _D), lambda b, m: (b, jnp.maximum(m - 1, 0), 0)),
            pl.BlockSpec((1, _MQ, _D), lambda b, m: (b, m, 0)),
            pl.BlockSpec((1, _MQ, _D), lambda b, m: (b, jnp.maximum(m - 1, 0), 0)),
            pl.BlockSpec((1, _MQ, _D), lambda b, m: (b, m, 0)),
            pl.BlockSpec((1, _TOPK, 1, _D), lambda b, m: (b, 0, 0, 0)),
            pl.BlockSpec((1, _TOPK, 1, _D), lambda b, m: (b, 0, 0, 0)),
            pl.BlockSpec((_D, _D), lambda b, m: (0, 0)),
            pl.BlockSpec((1, _D), lambda b, m: (0, 0)),
        ],
        out_specs=pl.BlockSpec((1, _MQ, _D), lambda b, m: (b, m, 0)),
        out_shape=jax.ShapeDtypeStruct((_B, _L, _D), jnp.float32),
        compiler_params=pltpu.CompilerParams(
            dimension_semantics=("parallel", "arbitrary")
        ),
    )(pos3, q3, k3, k3, v3, v3, kg4, vg4, wo, bo.reshape(1, _D))


def kernel(x, importance_mask, Wq, bq, Wk, bk, Wv, bv, Wo, bo):
    xf = x.reshape(_B * _L, _D)
    xbf = xf.astype(jnp.bfloat16)
    wq, wk, wv, wo = (w.astype(jnp.bfloat16) for w in (Wq, Wk, Wv, Wo))
    q2, k2, v2 = _qkv_proj(xbf, wq, wk, wv, bq, bk, bv, bm=512)

    idx_abs = _sc_topk(importance_mask.reshape(_B * _L))
    xg = _sc_gather_rows(xf, idx_abs)
    kg2, vg2 = _kv_glob_proj(xg.astype(jnp.bfloat16), wk, wv, bk, bv)
    topk = idx_abs.reshape(_B, 1, _TOPK)

    out = _attention(
        q2.reshape(_B, _L, _D), k2.reshape(_B, _L, _D),
        v2.reshape(_B, _L, _D),
        kg2.reshape(_B, _TOPK, 1, _D), vg2.reshape(_B, _TOPK, 1, _D),
        topk[0:1], wo, bo,
    )
    return out


# speedup vs baseline: 3.8481x; 1.1409x over previous
import dataclasses
import functools

import jax
import jax.numpy as jnp
from jax import lax
from jax.experimental import pallas as pl
from jax.experimental.pallas import tpu as pltpu
from jax.experimental.pallas import tpu_sc as plsc

_B, _L, _D = 2, 4096, 2048
_H, _DH = 16, 128
_TOPK, _WIN = 64, 128
_HALF = _WIN // 2
_SCALE = _DH ** -0.5

_MQ = 256
_NKV = _HALF + _MQ + _TOPK


def _proj_kernel_scaled(x_ref, w_ref, b_ref, o_ref):
    o_ref[...] = ((lax.dot_general(
        x_ref[...], w_ref[...], (((1,), (1,)), ((), ())),
        preferred_element_type=jnp.float32,
    ) + b_ref[...]) * _SCALE).astype(jnp.bfloat16)


def _proj(x2d, w, b1d, bm):
    m, k = x2d.shape
    n = w.shape[0]
    return pl.pallas_call(
        _proj_kernel_scaled,
        grid=(m // bm,),
        in_specs=[
            pl.BlockSpec((bm, k), lambda i: (i, 0)),
            pl.BlockSpec((n, k), lambda i: (0, 0)),
            pl.BlockSpec((1, n), lambda i: (0, 0)),
        ],
        out_specs=pl.BlockSpec((bm, n), lambda i: (i, 0)),
        out_shape=jax.ShapeDtypeStruct((m, n), jnp.bfloat16),
        compiler_params=pltpu.CompilerParams(
            dimension_semantics=("parallel",)
        ),
    )(x2d, w, b1d.reshape(1, n))


def _proj_g_kernel(x_ref, xg_ref, w_ref, b_ref, o_ref, og_ref):
    i = pl.program_id(0)
    last = pl.num_programs(0) - 1

    def proj(ref):
        return (lax.dot_general(
            ref[...], w_ref[...], (((1,), (1,)), ((), ())),
            preferred_element_type=jnp.float32,
        ) + b_ref[...]).astype(jnp.bfloat16)

    @pl.when(i < last)
    def _():
        o_ref[...] = proj(x_ref)

    @pl.when(i == last)
    def _():
        og_ref[...] = proj(xg_ref)


def _proj_g(x2d, xg, w, b1d, bm):
    m, k = x2d.shape
    n = w.shape[0]
    g = xg.shape[0]
    nm = m // bm
    return pl.pallas_call(
        _proj_g_kernel,
        grid=(nm + 1,),
        in_specs=[
            pl.BlockSpec((bm, k), lambda i: (jnp.minimum(i, nm - 1), 0)),
            pl.BlockSpec((g, k), lambda i: (0, 0)),
            pl.BlockSpec((n, k), lambda i: (0, 0)),
            pl.BlockSpec((1, n), lambda i: (0, 0)),
        ],
        out_specs=[
            pl.BlockSpec((bm, n), lambda i: (jnp.minimum(i, nm - 1), 0)),
            pl.BlockSpec((g, n), lambda i: (0, 0)),
        ],
        out_shape=[
            jax.ShapeDtypeStruct((m, n), jnp.bfloat16),
            jax.ShapeDtypeStruct((g, n), jnp.bfloat16),
        ],
        compiler_params=pltpu.CompilerParams(
            dimension_semantics=("arbitrary",)
        ),
    )(x2d, xg, w, b1d.reshape(1, n))


_NEG = float("-inf")


def _sc_topk(imf):
    ch = _L // 16
    nv = ch // 16
    mesh = plsc.VectorSubcoreMesh(core_axis_name="c", subcore_axis_name="s")
    cp = pltpu.CompilerParams()
    if "needs_layout_passes" in pltpu.CompilerParams.__dataclass_fields__:
        cp = dataclasses.replace(cp, needs_layout_passes=False)

    @functools.partial(
        pl.kernel, mesh=mesh, compiler_params=cp,
        out_type=jax.ShapeDtypeStruct((_B * _TOPK,), jnp.int32),
        scratch_types=[
            pltpu.VMEM((ch,), jnp.float32),
            pltpu.VMEM((_TOPK,), jnp.float32),
            pltpu.VMEM((_TOPK,), jnp.int32),
            pltpu.VMEM((16 * _TOPK,), jnp.float32),
            pltpu.VMEM((16 * _TOPK,), jnp.int32),
            pltpu.VMEM((_TOPK,), jnp.int32),
            pltpu.VMEM_SHARED((16 * _TOPK,), jnp.float32),
            pltpu.VMEM_SHARED((16 * _TOPK,), jnp.int32),
        ],
    )
    def k(im_hbm, out_hbm, chunk, lv, li, mv, mi, res, shv, shi):
        b = lax.axis_index("c")
        woff = lax.axis_index("s")
        base = b * _L + woff * ch
        pltpu.sync_copy(im_hbm.at[pl.ds(base, ch)], chunk)
        iota = lax.iota(jnp.int32, 16)
        big = jnp.int32(_B * _L)

        vs0 = tuple(chunk[pl.ds(i * 16, 16)] for i in range(nv))

        def p1_body(j, carry):
            vs = carry[0]
            rv = carry[1]
            ri = carry[2]
            m = vs[0]
            for i in range(1, nv):
                m = jnp.maximum(m, vs[i])
            mx = lax.reduce_max(m, axes=(0,))
            cand = jnp.where(vs[0] == mx, iota + (base + 0 * 16), big)
            for i in range(1, nv):
                cand = jnp.minimum(
                    cand, jnp.where(vs[i] == mx, iota + (base + i * 16), big))
            sel = lax.reduce_min(cand, axes=(0,))
            vs = tuple(
                jnp.where(iota + (base + i * 16) == sel, _NEG, vs[i])
                for i in range(nv))
            rv = tuple(
                jnp.where(iota + r * 16 == j, mx, rv[r])
                for r in range(_TOPK // 16))
            ri = tuple(
                jnp.where(iota + r * 16 == j, sel, ri[r])
                for r in range(_TOPK // 16))
            return (vs, rv, ri)

        z4f = tuple(jnp.zeros((16,), jnp.float32) for _ in range(_TOPK // 16))
        z4i = tuple(jnp.zeros((16,), jnp.int32) for _ in range(_TOPK // 16))
        _, rv, ri = lax.fori_loop(0, _TOPK, p1_body, (vs0, z4f, z4i))
        for r in range(_TOPK // 16):
            lv[pl.ds(r * 16, 16)] = rv[r]
            li[pl.ds(r * 16, 16)] = ri[r]
        pltpu.sync_copy(lv, shv.at[pl.ds(woff * _TOPK, _TOPK)])
        pltpu.sync_copy(li, shi.at[pl.ds(woff * _TOPK, _TOPK)])
        plsc.subcore_barrier()

        @pl.when(woff == 0)
        def _():
            pltpu.sync_copy(shv, mv)
            pltpu.sync_copy(shi, mi)
            nmv = 16 * _TOPK // 16

            def p2_body(j, _):
                m = mv[pl.ds(0, 16)]
                for i in range(1, nmv):
                    m = jnp.maximum(m, mv[pl.ds(i * 16, 16)])
                mx = lax.reduce_max(m, axes=(0,))
                cand = jnp.where(mv[pl.ds(0, 16)] == mx, mi[pl.ds(0, 16)], big)
                for i in range(1, nmv):
                    cand = jnp.minimum(
                        cand,
                        jnp.where(mv[pl.ds(i * 16, 16)] == mx,
                                  mi[pl.ds(i * 16, 16)], big))
                sel = lax.reduce_min(cand, axes=(0,))
                for i in range(nmv):
                    vv = mv[pl.ds(i * 16, 16)]
                    mv[pl.ds(i * 16, 16)] = jnp.where(
                        mi[pl.ds(i * 16, 16)] == sel, _NEG, vv)
                res[pl.ds(0, 16)] = jnp.where(
                    iota == j, sel, res[pl.ds(0, 16)])
                for r in range(1, _TOPK // 16):
                    res[pl.ds(r * 16, 16)] = jnp.where(
                        iota + r * 16 == j, sel, res[pl.ds(r * 16, 16)])
                return 0

            lax.fori_loop(0, _TOPK, p2_body, 0)
            pltpu.sync_copy(res, out_hbm.at[pl.ds(b * _TOPK, _TOPK)])

    return k(imf)


def _sc_gather_rows(x2d, idx_abs):
    nrows = _B * _TOPK
    per_w = 8
    nw_used = nrows // per_w
    mesh = plsc.VectorSubcoreMesh(core_axis_name="c", subcore_axis_name="s")

    @functools.partial(
        pl.kernel, mesh=mesh,
        out_type=jax.ShapeDtypeStruct((nrows, _D), jnp.float32),
        scratch_types=[
            pltpu.VMEM((per_w,), jnp.int32),
            pltpu.VMEM((per_w, _D), jnp.float32),
            pltpu.SemaphoreType.DMA,
        ],
    )
    def k(x_hbm, idx_hbm, out_hbm, idx_v, rows_v, sem):
        wid = lax.axis_index("s") * 2 + lax.axis_index("c")

        @pl.when(wid < nw_used)
        def _():
            base = wid * per_w
            pltpu.sync_copy(idx_hbm.at[pl.ds(base, per_w)], idx_v)
            pltpu.async_copy(x_hbm.at[idx_v], rows_v, sem).wait()
            pltpu.sync_copy(rows_v, out_hbm.at[pl.ds(base, per_w)])

    return k(x2d, idx_abs)


def _attn_kernel(pos_ref, q_ref, kp_ref, kc_ref, vp_ref, vc_ref,
                 kg_ref, vg_ref, wo_ref, bo_ref, o_ref):
    m_i = pl.program_id(1)
    q0 = m_i * _MQ
    ph0 = jnp.maximum(m_i - 1, 0) * _MQ + _MQ - _HALF

    colh = jax.lax.broadcasted_iota(jnp.int32, (1, _HALF), 1)
    col = jax.lax.broadcasted_iota(jnp.int32, (1, _MQ), 1)
    q_pos = q0 + jax.lax.broadcasted_iota(jnp.int32, (_MQ, 1), 0)
    kv_pos = jnp.concatenate(
        [ph0 + colh, q0 + col, pos_ref[0]], axis=1)
    nkcol = jax.lax.broadcasted_iota(jnp.int32, (1, _NKV), 1)
    is_prev = nkcol < _HALF
    is_glob = nkcol >= _HALF + _MQ
    win_start = (q_pos // _WIN) * _WIN
    local_ok = (kv_pos >= win_start - _HALF) & ((~is_prev) | (kv_pos < q0))
    vis = (kv_pos <= q_pos) & (is_glob | local_ok)
    mask_add = jnp.where(vis, 0.0, -jnp.inf)

    outs = []
    for h in range(_H):
        hs = slice(h * _DH, (h + 1) * _DH)
        q = q_ref[0, :, hs]
        kcat = jnp.concatenate(
            [kp_ref[0, :, hs], kc_ref[0, :, hs],
             kg_ref[0, :, 0, hs]], axis=0)
        vcat = jnp.concatenate(
            [vp_ref[0, :, hs], vc_ref[0, :, hs],
             vg_ref[0, :, 0, hs]], axis=0)
        s = lax.dot_general(
            q, kcat, (((1,), (1,)), ((), ())),
            preferred_element_type=jnp.float32,
        ) + mask_add
        mx = jnp.max(s, axis=1, keepdims=True)
        e = jnp.exp(s - mx)
        den = jnp.sum(e, axis=1, keepdims=True)
        o = lax.dot_general(
            e.astype(jnp.bfloat16), vcat, (((1,), (0,)), ((), ())),
            preferred_element_type=jnp.float32,
        ) / den
        outs.append(o.astype(jnp.bfloat16))
    o_all = jnp.concatenate(outs, axis=1)
    o_ref[0] = lax.dot_general(
        o_all, wo_ref[...], (((1,), (1,)), ((), ())),
        preferred_element_type=jnp.float32,
    ) + bo_ref[...]


def _attention(q3, k3, v3, kg4, vg4, pos3, wo, bo):
    nm = _L // _MQ
    return pl.pallas_call(
        _attn_kernel,
        grid=(_B, nm),
        in_specs=[
            pl.BlockSpec((1, 1, _TOPK), lambda b, m: (0, 0, 0)),
            pl.BlockSpec((1, _MQ, _D), lambda b, m: (b, m, 0)),
            pl.BlockSpec(
                (1, _HALF, _D),
                lambda b, m: (b, jnp.maximum(m * (_MQ // _HALF) - 1, 0), 0)),
            pl.BlockSpec((1, _MQ, _D), lambda b, m: (b, m, 0)),
            pl.BlockSpec(
                (1, _HALF, _D),
                lambda b, m: (b, jnp.maximum(m * (_MQ // _HALF) - 1, 0), 0)),
            pl.BlockSpec((1, _MQ, _D), lambda b, m: (b, m, 0)),
            pl.BlockSpec((1, _TOPK, 1, _D), lambda b, m: (b, 0, 0, 0)),
            pl.BlockSpec((1, _TOPK, 1, _D), lambda b, m: (b, 0, 0, 0)),
            pl.BlockSpec((_D, _D), lambda b, m: (0, 0)),
            pl.BlockSpec((1, _D), lambda b, m: (0, 0)),
        ],
        out_specs=pl.BlockSpec((1, _MQ, _D), lambda b, m: (b, m, 0)),
        out_shape=jax.ShapeDtypeStruct((_B, _L, _D), jnp.float32),
        compiler_params=pltpu.CompilerParams(
            dimension_semantics=("parallel", "arbitrary")
        ),
    )(pos3, q3, k3, k3, v3, v3, kg4, vg4, wo, bo.reshape(1, _D))


def kernel(x, importance_mask, Wq, bq, Wk, bk, Wv, bv, Wo, bo):
    xf = x.reshape(_B * _L, _D)
    wo = Wo.astype(jnp.bfloat16)
    idx_abs = _sc_topk(importance_mask.reshape(_B * _L))
    xg = _sc_gather_rows(xf, idx_abs)

    q2 = _proj(xf, Wq, bq, bm=512)
    k2, kg2 = _proj_g(xf, xg, Wk, bk, bm=512)
    v2, vg2 = _proj_g(xf, xg, Wv, bv, bm=512)
    topk = idx_abs.reshape(_B, 1, _TOPK)

    out = _attention(
        q2.reshape(_B, _L, _D), k2.reshape(_B, _L, _D),
        v2.reshape(_B, _L, _D),
        kg2.reshape(_B, _TOPK, 1, _D), vg2.reshape(_B, _TOPK, 1, _D),
        topk[0:1], wo, bo,
    )
    return out
